# baseline (speedup 1.0000x reference)
import jax
import jax.numpy as jnp
from jax.experimental import pallas as pl

_B, _C, _H, _W = 128, 768, 14, 14
_HW = _H * _W
_BB = 8


def _fused(x_ref, cr_ref, snr_ref, w1_ref, b1_ref, w2t_ref, b2_ref,
           ch0_ref, wctc_ref, wcts_ref, wst_ref, wot_ref, wclst_ref,
           bcls_ref, out_ref):
    sem = jnp.sum(x_ref[...], axis=2) * (1.0 / _HW)

    h1 = jnp.maximum(snr_ref[0, 0] * w1_ref[...] + b1_ref[...], 0.0)
    sv = jnp.maximum(
        jnp.dot(h1, w2t_ref[...], preferred_element_type=jnp.float32)
        + b2_ref[...], 0.0)
    contrib = (
        jnp.dot(ch0_ref[...], wctc_ref[...], preferred_element_type=jnp.float32)
        + jnp.dot(sv, wcts_ref[...], preferred_element_type=jnp.float32))

    hid = jnp.maximum(
        jnp.dot(sem, wst_ref[...], preferred_element_type=jnp.float32)
        + contrib, 0.0)
    wts = jax.nn.sigmoid(
        jnp.dot(hid, wot_ref[...], preferred_element_type=jnp.float32))

    cr_c = jnp.clip(cr_ref[...], 0.001, 1.0)
    k = jnp.clip(jnp.round(cr_c * _C), 1.0, float(_C)).astype(jnp.int32)

    bits = jax.lax.bitcast_convert_type(wts, jnp.int32)

    def body(i, t):
        cand = t | (jnp.int32(1) << (jnp.int32(30) - i))
        cnt = jnp.sum((bits >= cand).astype(jnp.int32), axis=1, keepdims=True)
        return jnp.where(cnt >= k, cand, t)

    t = jax.lax.fori_loop(0, 31, body, jnp.zeros_like(k))
    mask = (bits >= t).astype(jnp.float32)

    pooled = sem * wts * mask
    out_ref[...] = (
        jnp.dot(pooled, wclst_ref[...], preferred_element_type=jnp.float32)
        + bcls_ref[...])


def kernel(x, snr_db, cr, channel_embed, snr_w1, snr_b1, snr_w2, snr_b2,
           Ws, Wc, Wo, Wcls, bcls):
    B, C, H, W = x.shape
    hw = H * W
    xr = x.reshape(B, C, hw)
    crr = cr.reshape(B, 1)
    snr_sc = (jnp.asarray(snr_db, dtype=x.dtype) / 28.0).reshape(1, 1)
    w1r = snr_w1.T
    b1r = snr_b1.reshape(1, -1)
    w2t = snr_w2.T
    b2r = snr_b2.reshape(1, -1)
    ch0 = channel_embed[0].reshape(1, -1)
    e = channel_embed.shape[1]
    wct = Wc.T
    wctc, wcts = wct[:e], wct[e:]
    wst = Ws.T
    wot = Wo.T
    wclst = Wcls.T
    bclsr = bcls.reshape(1, -1)
    n_cls = Wcls.shape[0]
    hidden = Ws.shape[0]

    grid = (B // _BB,)
    const = lambda shape: pl.BlockSpec(shape, lambda i: (0,) * len(shape))
    return pl.pallas_call(
        _fused,
        grid=grid,
        in_specs=[
            pl.BlockSpec((_BB, C, hw), lambda i: (i, 0, 0)),
            pl.BlockSpec((_BB, 1), lambda i: (i, 0)),
            const((1, 1)),
            const((1, e)),
            const((1, e)),
            const((e, e)),
            const((1, e)),
            const((1, e)),
            const((e, hidden)),
            const((e, hidden)),
            const((C, hidden)),
            const((hidden, C)),
            const((C, n_cls)),
            const((1, n_cls)),
        ],
        out_specs=pl.BlockSpec((_BB, n_cls), lambda i: (i, 0)),
        out_shape=jax.ShapeDtypeStruct((B, n_cls), x.dtype),
    )(xr, crr, snr_sc, w1r, b1r, w2t, b2r, ch0, wctc, wcts, wst, wot,
      wclst, bclsr)
